# Initial kernel scaffold; baseline (speedup 1.0000x reference)
import jax
import jax.numpy as jnp
from jax.experimental import pallas as pl


def kernel(input_, weight):
    raise NotImplementedError("write your pallas kernel here")


# speedup vs baseline: 1.6824x; 1.6824x over previous
import functools

import jax
import jax.numpy as jnp
from jax import lax
from jax.experimental import pallas as pl
from jax.experimental.pallas import tpu as pltpu
from jax.experimental.pallas import tpu_sc as plsc

_NC = 2
_NS = 16
_NW = _NC * _NS
_CHUNK = 128


@functools.partial(jax.jit, static_argnums=(2, 3))
def _sc_gather(idx, weight, n_chunks, d):
  mesh = plsc.VectorSubcoreMesh(
      core_axis_name="c", subcore_axis_name="s", num_cores=_NC,
      num_subcores=_NS)
  n_total = _NW * n_chunks * _CHUNK

  @functools.partial(
      pl.kernel,
      out_type=jax.ShapeDtypeStruct((n_total, d), jnp.float32),
      mesh=mesh,
      scratch_types=[
          pltpu.VMEM((n_chunks, _CHUNK), jnp.int32),
          pltpu.VMEM((_CHUNK, d), jnp.float32),
          pltpu.SemaphoreType.DMA,
      ],
      compiler_params=pltpu.CompilerParams(use_tc_tiling_on_sc=False),
  )
  def grab(idx_hbm, w_hbm, out_hbm, idx_v, rows_v, gsem):
    wid = lax.axis_index("s") * _NC + lax.axis_index("c")
    pltpu.sync_copy(idx_hbm.at[wid], idx_v)
    base = wid * (n_chunks * _CHUNK)

    @pl.loop(0, n_chunks)
    def _(j):
      pltpu.async_copy(w_hbm.at[idx_v.at[j]], rows_v, gsem).wait()
      pltpu.sync_copy(rows_v, out_hbm.at[pl.ds(base + j * _CHUNK, _CHUNK)])

  return grab(idx, weight)


def kernel(input_, weight):
  b, h = input_.shape
  _, d = weight.shape
  n = b * h
  n_chunks = n // (_NW * _CHUNK)
  idx = input_.reshape(_NW, n_chunks, _CHUNK)
  out = _sc_gather(idx, weight, n_chunks, d)
  return out.reshape(b, h, d)


# speedup vs baseline: 1.8702x; 1.1116x over previous
import functools

import jax
import jax.numpy as jnp
from jax import lax
from jax.experimental import pallas as pl
from jax.experimental.pallas import tpu as pltpu
from jax.experimental.pallas import tpu_sc as plsc

_NC = 2
_NS = 16
_NW = _NC * _NS
_CHUNK = 128


_NBUF = 8


@functools.partial(jax.jit, static_argnums=(2, 3))
def _sc_gather(idx, weight, n_chunks, d):
  mesh = plsc.VectorSubcoreMesh(
      core_axis_name="c", subcore_axis_name="s", num_cores=_NC,
      num_subcores=_NS)
  n_total = _NW * n_chunks * _CHUNK
  n_outer = n_chunks // _NBUF

  @functools.partial(
      pl.kernel,
      out_type=jax.ShapeDtypeStruct((n_total, d), jnp.float32),
      mesh=mesh,
      scratch_types=[
          pltpu.VMEM((n_chunks, _CHUNK), jnp.int32),
          pltpu.VMEM((_NBUF, _CHUNK, d), jnp.float32),
          [pltpu.SemaphoreType.DMA] * _NBUF,
          [pltpu.SemaphoreType.DMA] * _NBUF,
      ],
      compiler_params=pltpu.CompilerParams(use_tc_tiling_on_sc=False),
  )
  def grab(idx_hbm, w_hbm, out_hbm, idx_v, rows_v, gsems, ssems):
    wid = lax.axis_index("s") * _NC + lax.axis_index("c")
    pltpu.sync_copy(idx_hbm.at[wid], idx_v)
    base = wid * (n_chunks * _CHUNK)

    for b in range(_NBUF):
      pltpu.async_copy(w_hbm.at[idx_v.at[b]], rows_v.at[b], gsems[b])

    @pl.loop(0, n_outer)
    def _(g):
      for b in range(_NBUF):
        j = g * _NBUF + b
        pltpu.make_async_copy(
            w_hbm.at[idx_v.at[j]], rows_v.at[b], gsems[b]).wait()
        pltpu.async_copy(
            rows_v.at[b], out_hbm.at[pl.ds(base + j * _CHUNK, _CHUNK)],
            ssems[b])
      for b in range(_NBUF):
        j = g * _NBUF + b
        nxt = j + _NBUF

        @pl.when(nxt < n_chunks)
        def _():
          pltpu.make_async_copy(
              rows_v.at[b], out_hbm.at[pl.ds(base + j * _CHUNK, _CHUNK)],
              ssems[b]).wait()
          pltpu.async_copy(w_hbm.at[idx_v.at[nxt]], rows_v.at[b], gsems[b])

    for b in range(_NBUF):
      j = (n_outer - 1) * _NBUF + b
      pltpu.make_async_copy(
          rows_v.at[b], out_hbm.at[pl.ds(base + j * _CHUNK, _CHUNK)],
          ssems[b]).wait()

  return grab(idx, weight)


def kernel(input_, weight):
  b, h = input_.shape
  _, d = weight.shape
  n = b * h
  n_chunks = n // (_NW * _CHUNK)
  idx = input_.reshape(_NW, n_chunks, _CHUNK)
  out = _sc_gather(idx, weight, n_chunks, d)
  return out.reshape(b, h, d)
